# baseline (speedup 1.0000x reference)
import functools

import jax
import jax.numpy as jnp
from jax.experimental import pallas as pl
from jax.experimental.pallas import tpu as pltpu


def _pick_bm(n):
    for bm in (200, 128, 80, 40, 8):
        if n % bm == 0:
            return bm
    return n


def _ln_leaky(h, g_ref, beta_ref):
    m = jnp.mean(h, axis=-1, keepdims=True)
    v = jnp.mean((h - m) ** 2, axis=-1, keepdims=True)
    h = (h - m) * jax.lax.rsqrt(v + 1e-5) * g_ref[:] + beta_ref[:]
    return jnp.where(h >= 0, h, 0.01 * h)


def _fused_body(x_ref, a0_ref, a1_ref, w1_ref, b1_ref, g1_ref, beta1_ref,
                w2_ref, b2_ref, g2_ref, beta2_ref, wl_ref, bl_ref,
                out_ref, p_ref, h_ref, q_ref, *, bm):
    i = pl.program_id(0)
    nb = pl.num_programs(0) // 2

    @pl.when(i == 0)
    def _():
        p_ref[:] = jnp.dot(x_ref[:], w1_ref[:],
                           preferred_element_type=jnp.float32
                           ).astype(jnp.bfloat16)

    @pl.when(i < nb)
    def _():
        a = a0_ref[:].astype(jnp.bfloat16)
        h = jnp.dot(a, p_ref[:],
                    preferred_element_type=jnp.float32) + b1_ref[:]
        h_ref[pl.ds(i * bm, bm), :] = _ln_leaky(
            h, g1_ref, beta1_ref).astype(jnp.bfloat16)

    @pl.when(i == nb)
    def _():
        q_ref[:] = jnp.dot(h_ref[:].astype(jnp.float32), w2_ref[:],
                           preferred_element_type=jnp.float32
                           ).astype(jnp.bfloat16)

    @pl.when(i >= nb)
    def _():
        a = a1_ref[:].astype(jnp.bfloat16)
        g = jnp.dot(a, q_ref[:],
                    preferred_element_type=jnp.float32) + b2_ref[:]
        g = _ln_leaky(g, g2_ref, beta2_ref)
        z = jnp.dot(g, wl_ref[:],
                    preferred_element_type=jnp.float32) + bl_ref[:]
        zmax = jnp.max(z, axis=-1, keepdims=True)
        z = z - zmax
        out_ref[:] = z - jnp.log(jnp.sum(jnp.exp(z), axis=-1, keepdims=True))


@jax.jit
def kernel(x, A0, A1, W1, b1, g1, beta1, W2, b2, g2, beta2, Wl, bl):
    n, fan_in = x.shape
    fan_mid = W1.shape[1]
    fm2 = W2.shape[1]
    fan_out = Wl.shape[1]
    bm = _pick_bm(n)
    nb = n // bm

    full = lambda r, c: pl.BlockSpec((r, c), lambda i: (0, 0))

    out = pl.pallas_call(
        functools.partial(_fused_body, bm=bm),
        grid=(2 * nb,),
        in_specs=[
            full(n, fan_in),
            pl.BlockSpec((bm, n), lambda i: (jnp.minimum(i, nb - 1), 0)),
            pl.BlockSpec((bm, n), lambda i: (jnp.maximum(i - nb, 0), 0)),
            full(fan_in, fan_mid),
            full(1, fan_mid), full(1, fan_mid), full(1, fan_mid),
            full(fan_mid, fm2),
            full(1, fm2), full(1, fm2), full(1, fm2),
            full(fm2, fan_out),
            full(1, fan_out),
        ],
        out_specs=pl.BlockSpec((bm, fan_out),
                               lambda i: (jnp.maximum(i - nb, 0), 0)),
        out_shape=jax.ShapeDtypeStruct((n, fan_out), jnp.float32),
        scratch_shapes=[
            pltpu.VMEM((n, fan_mid), jnp.bfloat16),
            pltpu.VMEM((n, fan_mid), jnp.bfloat16),
            pltpu.VMEM((n, fm2), jnp.bfloat16),
        ],
        compiler_params=pltpu.CompilerParams(
            dimension_semantics=("arbitrary",),
            vmem_limit_bytes=63 * 1024 * 1024),
    )(x, A0, A1, W1, b1.reshape(1, -1), g1.reshape(1, -1), beta1.reshape(1, -1),
      W2, b2.reshape(1, -1), g2.reshape(1, -1), beta2.reshape(1, -1),
      Wl, bl.reshape(1, -1))

    return out


# speedup vs baseline: 1.0222x; 1.0222x over previous
import functools

import jax
import jax.numpy as jnp
from jax.experimental import pallas as pl
from jax.experimental.pallas import tpu as pltpu


def _pick_bm(n):
    for bm in (400, 256, 208, 128, 80, 16):
        if n % bm == 0:
            return bm
    return n


def _proj_body(a_ref, w_ref, o_ref):
    o_ref[:] = jnp.dot(a_ref[:].astype(jnp.float32), w_ref[:],
                       preferred_element_type=jnp.float32
                       ).astype(jnp.bfloat16)


def _proj(a, w):
    n = a.shape[0]
    return pl.pallas_call(
        _proj_body,
        out_shape=jax.ShapeDtypeStruct((n, w.shape[1]), jnp.bfloat16),
    )(a, w)


def _ln_leaky(h, g_ref, beta_ref):
    m = jnp.mean(h, axis=-1, keepdims=True)
    v = jnp.mean((h - m) ** 2, axis=-1, keepdims=True)
    h = (h - m) * jax.lax.rsqrt(v + 1e-5) * g_ref[:] + beta_ref[:]
    return jnp.where(h >= 0, h, 0.01 * h)


def _layer1_body(a_ref, p_ref, b1_ref, g1_ref, beta1_ref, out_ref):
    a = a_ref[:].astype(jnp.bfloat16)
    h = jnp.dot(a, p_ref[:], preferred_element_type=jnp.float32) + b1_ref[:]
    out_ref[:] = _ln_leaky(h, g1_ref, beta1_ref).astype(jnp.bfloat16)


def _layer2_body(a_ref, q_ref, b2_ref, g2_ref, beta2_ref, wl_ref, bl_ref,
                 out_ref):
    a = a_ref[:].astype(jnp.bfloat16)
    g = jnp.dot(a, q_ref[:], preferred_element_type=jnp.float32) + b2_ref[:]
    g = _ln_leaky(g, g2_ref, beta2_ref)
    z = jnp.dot(g, wl_ref[:], preferred_element_type=jnp.float32) + bl_ref[:]
    zmax = jnp.max(z, axis=-1, keepdims=True)
    z = z - zmax
    out_ref[:] = z - jnp.log(jnp.sum(jnp.exp(z), axis=-1, keepdims=True))


@jax.jit
def kernel(x, A0, A1, W1, b1, g1, beta1, W2, b2, g2, beta2, Wl, bl):
    n, fan_in = x.shape
    fan_mid = W1.shape[1]
    fm2 = W2.shape[1]
    fan_out = Wl.shape[1]
    bm = _pick_bm(n)
    grid = (n // bm,)

    full = lambda r, c: pl.BlockSpec((r, c), lambda i: (0, 0))
    rows = lambda c: pl.BlockSpec((bm, c), lambda i: (i, 0))
    params = pltpu.CompilerParams(dimension_semantics=("parallel",))

    p = _proj(x, W1)

    h = pl.pallas_call(
        _layer1_body,
        grid=grid,
        in_specs=[
            rows(n),
            full(n, fan_mid),
            full(1, fan_mid), full(1, fan_mid), full(1, fan_mid),
        ],
        out_specs=rows(fan_mid),
        out_shape=jax.ShapeDtypeStruct((n, fan_mid), jnp.bfloat16),
        compiler_params=params,
    )(A0, p, b1.reshape(1, -1), g1.reshape(1, -1), beta1.reshape(1, -1))

    q = _proj(h, W2)

    out = pl.pallas_call(
        _layer2_body,
        grid=grid,
        in_specs=[
            rows(n),
            full(n, fm2),
            full(1, fm2), full(1, fm2), full(1, fm2),
            full(fm2, fan_out),
            full(1, fan_out),
        ],
        out_specs=rows(fan_out),
        out_shape=jax.ShapeDtypeStruct((n, fan_out), jnp.float32),
        compiler_params=params,
    )(A1, q, b2.reshape(1, -1), g2.reshape(1, -1), beta2.reshape(1, -1),
      Wl, bl.reshape(1, -1))

    return out


# speedup vs baseline: 1.0539x; 1.0310x over previous
import jax
import jax.numpy as jnp
from jax.experimental import pallas as pl
from jax.experimental.pallas import tpu as pltpu


def _pick_bm(n):
    for bm in (400, 256, 208, 128, 80, 16):
        if n % bm == 0:
            return bm
    return n


def _ln_leaky(h, g_ref, beta_ref):
    m = jnp.mean(h, axis=-1, keepdims=True)
    v = jnp.mean((h - m) ** 2, axis=-1, keepdims=True)
    h = (h - m) * jax.lax.rsqrt(v + 1e-5) * g_ref[:] + beta_ref[:]
    return jnp.where(h >= 0, h, 0.01 * h)


def _layer1_body(x_ref, a_ref, w1_ref, b1_ref, g1_ref, beta1_ref,
                 out_ref, p_ref):
    @pl.when(pl.program_id(0) == 0)
    def _():
        p_ref[:] = jnp.dot(x_ref[:].astype(jnp.bfloat16),
                           w1_ref[:].astype(jnp.bfloat16),
                           preferred_element_type=jnp.float32
                           ).astype(jnp.bfloat16)

    a = a_ref[:].astype(jnp.bfloat16)
    h = jnp.dot(a, p_ref[:], preferred_element_type=jnp.float32) + b1_ref[:]
    out_ref[:] = _ln_leaky(h, g1_ref, beta1_ref).astype(jnp.bfloat16)


def _layer2_body(h_ref, a_ref, w2_ref, b2_ref, g2_ref, beta2_ref,
                 wl_ref, bl_ref, out_ref, q_ref):
    @pl.when(pl.program_id(0) == 0)
    def _():
        q_ref[:] = jnp.dot(h_ref[:], w2_ref[:].astype(jnp.bfloat16),
                           preferred_element_type=jnp.float32
                           ).astype(jnp.bfloat16)

    a = a_ref[:].astype(jnp.bfloat16)
    g = jnp.dot(a, q_ref[:], preferred_element_type=jnp.float32) + b2_ref[:]
    g = _ln_leaky(g, g2_ref, beta2_ref)
    z = jnp.dot(g, wl_ref[:], preferred_element_type=jnp.float32) + bl_ref[:]
    zmax = jnp.max(z, axis=-1, keepdims=True)
    z = z - zmax
    out_ref[:] = z - jnp.log(jnp.sum(jnp.exp(z), axis=-1, keepdims=True))


@jax.jit
def kernel(x, A0, A1, W1, b1, g1, beta1, W2, b2, g2, beta2, Wl, bl):
    n, fan_in = x.shape
    fan_mid = W1.shape[1]
    fm2 = W2.shape[1]
    fan_out = Wl.shape[1]
    bm = _pick_bm(n)
    grid = (n // bm,)

    full = lambda r, c: pl.BlockSpec((r, c), lambda i: (0, 0))
    rows = lambda c: pl.BlockSpec((bm, c), lambda i: (i, 0))
    params = pltpu.CompilerParams(dimension_semantics=("arbitrary",))

    h = pl.pallas_call(
        _layer1_body,
        grid=grid,
        in_specs=[
            full(n, fan_in),
            rows(n),
            full(fan_in, fan_mid),
            full(1, fan_mid), full(1, fan_mid), full(1, fan_mid),
        ],
        out_specs=rows(fan_mid),
        out_shape=jax.ShapeDtypeStruct((n, fan_mid), jnp.bfloat16),
        scratch_shapes=[pltpu.VMEM((n, fan_mid), jnp.bfloat16)],
        compiler_params=params,
    )(x, A0, W1, b1.reshape(1, -1), g1.reshape(1, -1), beta1.reshape(1, -1))

    out = pl.pallas_call(
        _layer2_body,
        grid=grid,
        in_specs=[
            full(n, fan_mid),
            rows(n),
            full(fan_mid, fm2),
            full(1, fm2), full(1, fm2), full(1, fm2),
            full(fm2, fan_out),
            full(1, fan_out),
        ],
        out_specs=rows(fan_out),
        out_shape=jax.ShapeDtypeStruct((n, fan_out), jnp.float32),
        scratch_shapes=[pltpu.VMEM((n, fm2), jnp.bfloat16)],
        compiler_params=params,
    )(h, A1, W2, b2.reshape(1, -1), g2.reshape(1, -1), beta2.reshape(1, -1),
      Wl, bl.reshape(1, -1))

    return out
